# baseline (speedup 1.0000x reference)
import functools

import jax
import jax.numpy as jnp
from jax.experimental import pallas as pl
from jax.experimental.pallas import tpu as pltpu


def _fc1_pack_kernel(x_ref, w_ref, b_ref, o_ref):
    tn = x_ref.shape[0]
    x = x_ref[...].astype(jnp.bfloat16)
    w = w_ref[...].astype(jnp.bfloat16)
    acc = jax.lax.dot_general(
        x, w, (((1,), (1,)), ((), ())),
        preferred_element_type=jnp.float32)
    bf = (acc + b_ref[...]).astype(jnp.bfloat16)
    o_ref[...] = pltpu.bitcast(bf.reshape(tn, 8, bf.shape[1] // 8),
                               jnp.int32)


def _pack_kernel(x_ref, o_ref):
    tn = x_ref.shape[0]
    bf = x_ref[...].astype(jnp.bfloat16)
    o_ref[...] = pltpu.bitcast(bf.reshape(tn, 8, bf.shape[1] // 8),
                               jnp.int32)


def _prep_kernel(x2_ref, w_ref, b_ref, x1_ref, o2_ref, o1_ref):
    _fc1_pack_kernel(x2_ref, w_ref, b_ref, o2_ref)
    _pack_kernel(x1_ref, o1_ref)


def _dist_kernel(ii_ref, jj_ref, e1_ref, e2_ref, ones_ref, out_ref,
                 sq_ref, acc_ref, *, tile_p, unroll):
    t = pl.program_id(1)

    @pl.when(t == 0)
    def _init():
        acc_ref[...] = jnp.zeros_like(acc_ref)

    def chunk(cstep, carry):
        base = cstep * unroll
        loads = []
        for u in range(unroll):
            i = ii_ref[0, 0, base + u]
            j = jj_ref[0, 0, base + u]
            loads.append((e1_ref[i], e2_ref[j]))
        for u, (ai, bj) in enumerate(loads):
            a = pltpu.bitcast(ai, jnp.bfloat16)
            b = pltpu.bitcast(bj, jnp.bfloat16)
            d = (a - b).astype(jnp.float32)
            sq_ref[base + u] = d * d
        return carry

    jax.lax.fori_loop(0, tile_p // unroll, chunk, jnp.int32(0))

    def bblock(bstep, carry):
        x = sq_ref[pl.ds(bstep * 128, 128)]
        r = jnp.sum(x, axis=1).astype(jnp.bfloat16)
        s = jax.lax.dot_general(
            r, ones_ref[...], (((1,), (0,)), ((), ())),
            preferred_element_type=jnp.float32)
        dist = jnp.sqrt(s).reshape(16, 8, 128)
        acc_ref[...] += jnp.sum(dist, axis=0)
        return carry

    jax.lax.fori_loop(0, tile_p // 128, bblock, jnp.int32(0))

    @pl.when(t == pl.num_programs(1) - 1)
    def _finalize():
        out_ref[0, 0, 0] = jnp.sum(acc_ref[...]) * jnp.float32(1.0 / 128.0)


def kernel(embedding_1, embedding_2, observed_anchors_p, fc1_w, fc1_b):
    n1, d = embedding_1.shape
    n2, _ = embedding_2.shape
    p = observed_anchors_p.shape[0]

    tile_n = 512
    tn = min(n1, n2, tile_n)
    bias_2d = fc1_b.reshape(1, d)
    if n1 == n2:
        e2_pack, e1_pack = pl.pallas_call(
            _prep_kernel,
            grid=(pl.cdiv(n2, tn),),
            in_specs=[
                pl.BlockSpec((tn, d), lambda i: (i, 0)),
                pl.BlockSpec((d, d), lambda i: (0, 0)),
                pl.BlockSpec((1, d), lambda i: (0, 0)),
                pl.BlockSpec((tn, d), lambda i: (i, 0)),
            ],
            out_specs=[
                pl.BlockSpec((tn, 4, d // 8), lambda i: (i, 0, 0)),
                pl.BlockSpec((tn, 4, d // 8), lambda i: (i, 0, 0)),
            ],
            out_shape=[
                jax.ShapeDtypeStruct((n2, 4, d // 8), jnp.int32),
                jax.ShapeDtypeStruct((n1, 4, d // 8), jnp.int32),
            ],
            compiler_params=pltpu.CompilerParams(
                dimension_semantics=("parallel",),
            ),
            cost_estimate=pl.CostEstimate(
                flops=2 * n2 * d * d + n2 * d,
                transcendentals=0,
                bytes_accessed=(2 * n2 * d + n2 * d + d * d) * 4,
            ),
        )(embedding_2, fc1_w, bias_2d, embedding_1)
    else:
        e2_pack = pl.pallas_call(
            _fc1_pack_kernel,
            grid=(pl.cdiv(n2, tn),),
            in_specs=[
                pl.BlockSpec((tn, d), lambda i: (i, 0)),
                pl.BlockSpec((d, d), lambda i: (0, 0)),
                pl.BlockSpec((1, d), lambda i: (0, 0)),
            ],
            out_specs=pl.BlockSpec((tn, 4, d // 8), lambda i: (i, 0, 0)),
            out_shape=jax.ShapeDtypeStruct((n2, 4, d // 8), jnp.int32),
            compiler_params=pltpu.CompilerParams(
                dimension_semantics=("parallel",),
            ),
        )(embedding_2, fc1_w, bias_2d)
        e1_pack = pl.pallas_call(
            _pack_kernel,
            grid=(pl.cdiv(n1, tn),),
            in_specs=[pl.BlockSpec((tn, d), lambda i: (i, 0))],
            out_specs=pl.BlockSpec((tn, 4, d // 8), lambda i: (i, 0, 0)),
            out_shape=jax.ShapeDtypeStruct((n1, 4, d // 8), jnp.int32),
            compiler_params=pltpu.CompilerParams(
                dimension_semantics=("parallel",),
            ),
        )(embedding_1)

    ones_mat = jnp.ones((d // 8, 128), jnp.bfloat16)

    n_cores = 2
    per_core_p = p // n_cores
    tile_p = per_core_p if per_core_p <= 2048 else 2048
    unroll = 128 if tile_p % 128 == 0 else 1
    ii = observed_anchors_p[:, 0]
    jj = observed_anchors_p[:, 1]
    num_tiles = per_core_p // tile_p
    ii3 = ii.reshape(n_cores * num_tiles, 1, tile_p)
    jj3 = jj.reshape(n_cores * num_tiles, 1, tile_p)

    dk = functools.partial(_dist_kernel, tile_p=tile_p, unroll=unroll)
    partials = pl.pallas_call(
        dk,
        grid=(n_cores, num_tiles),
        in_specs=[
            pl.BlockSpec((1, 1, tile_p),
                         lambda c, t, _nt=num_tiles: (c * _nt + t, 0, 0),
                         memory_space=pltpu.MemorySpace.SMEM),
            pl.BlockSpec((1, 1, tile_p),
                         lambda c, t, _nt=num_tiles: (c * _nt + t, 0, 0),
                         memory_space=pltpu.MemorySpace.SMEM),
            pl.BlockSpec((n1, 4, d // 8), lambda c, t: (0, 0, 0)),
            pl.BlockSpec((n2, 4, d // 8), lambda c, t: (0, 0, 0)),
            pl.BlockSpec((d // 8, 128), lambda c, t: (0, 0)),
        ],
        out_specs=pl.BlockSpec((1, 1, 1), lambda c, t: (c, 0, 0),
                               memory_space=pltpu.MemorySpace.SMEM),
        out_shape=jax.ShapeDtypeStruct((n_cores, 1, 1), jnp.float32),
        scratch_shapes=[
            pltpu.VMEM((tile_p, 8, d // 8), jnp.float32),
            pltpu.VMEM((8, 128), jnp.float32),
        ],
        compiler_params=pltpu.CompilerParams(
            dimension_semantics=("parallel", "arbitrary"),
        ),
        cost_estimate=pl.CostEstimate(
            flops=4 * p * d,
            transcendentals=p,
            bytes_accessed=(n1 * d // 2 + n2 * d // 2 + 2 * p) * 4,
        ),
    )(ii3, jj3, e1_pack, e2_pack, ones_mat)

    return jnp.sum(partials) * jnp.float32(1.0 / p)


# speedup vs baseline: 1.7315x; 1.7315x over previous
import functools

import jax
import jax.numpy as jnp
from jax.experimental import pallas as pl
from jax.experimental.pallas import tpu as pltpu


def _fc1_pack_kernel(x_ref, w_ref, b_ref, o_ref):
    tn = x_ref.shape[0]
    x = x_ref[...].astype(jnp.bfloat16)
    w = w_ref[...].astype(jnp.bfloat16)
    acc = jax.lax.dot_general(
        x, w, (((1,), (1,)), ((), ())),
        preferred_element_type=jnp.float32)
    bf = (acc + b_ref[...]).astype(jnp.bfloat16)
    o_ref[...] = pltpu.bitcast(bf.reshape(tn, 8, bf.shape[1] // 8),
                               jnp.int32)


def _pack_kernel(x_ref, o_ref):
    tn = x_ref.shape[0]
    bf = x_ref[...].astype(jnp.bfloat16)
    o_ref[...] = pltpu.bitcast(bf.reshape(tn, 8, bf.shape[1] // 8),
                               jnp.int32)


def _prep_kernel(x2_ref, w_ref, b_ref, x1_ref, o2_ref, o1_ref):
    _fc1_pack_kernel(x2_ref, w_ref, b_ref, o2_ref)
    _pack_kernel(x1_ref, o1_ref)


def _dist_kernel(ii_ref, jj_ref, e1_ref, e2_ref, ones_ref, out_ref,
                 sq_ref, acc_ref, *, tile_p, unroll):
    t = pl.program_id(1)

    @pl.when(t == 0)
    def _init():
        acc_ref[...] = jnp.zeros_like(acc_ref)

    def chunk(cstep, carry):
        base = cstep * unroll
        loads = []
        for u in range(unroll):
            i = ii_ref[0, 0, base + u]
            j = jj_ref[0, 0, base + u]
            loads.append((e1_ref[i], e2_ref[j]))
        for u, (ai, bj) in enumerate(loads):
            a = pltpu.bitcast(ai, jnp.bfloat16)
            b = pltpu.bitcast(bj, jnp.bfloat16)
            d = (a - b).astype(jnp.float32)
            row = cstep * (unroll // 8) + u // 8
            sq_ref[row, u % 8, :] = jnp.sum(d * d, axis=0)
        return carry

    jax.lax.fori_loop(0, tile_p // unroll, chunk, jnp.int32(0))

    y = sq_ref[...].reshape(tile_p, sq_ref.shape[2]).astype(jnp.bfloat16)
    s = jax.lax.dot_general(
        y, ones_ref[...], (((1,), (0,)), ((), ())),
        preferred_element_type=jnp.float32)
    dist = jnp.sqrt(s).reshape(tile_p // 8, 8, 128)
    acc_ref[...] += jnp.sum(dist, axis=0)

    @pl.when(t == pl.num_programs(1) - 1)
    def _finalize():
        out_ref[0, 0, 0] = jnp.sum(acc_ref[...]) * jnp.float32(1.0 / 128.0)


def kernel(embedding_1, embedding_2, observed_anchors_p, fc1_w, fc1_b):
    n1, d = embedding_1.shape
    n2, _ = embedding_2.shape
    p = observed_anchors_p.shape[0]

    tile_n = 512
    tn = min(n1, n2, tile_n)
    bias_2d = fc1_b.reshape(1, d)
    if n1 == n2:
        e2_pack, e1_pack = pl.pallas_call(
            _prep_kernel,
            grid=(pl.cdiv(n2, tn),),
            in_specs=[
                pl.BlockSpec((tn, d), lambda i: (i, 0)),
                pl.BlockSpec((d, d), lambda i: (0, 0)),
                pl.BlockSpec((1, d), lambda i: (0, 0)),
                pl.BlockSpec((tn, d), lambda i: (i, 0)),
            ],
            out_specs=[
                pl.BlockSpec((tn, 4, d // 8), lambda i: (i, 0, 0)),
                pl.BlockSpec((tn, 4, d // 8), lambda i: (i, 0, 0)),
            ],
            out_shape=[
                jax.ShapeDtypeStruct((n2, 4, d // 8), jnp.int32),
                jax.ShapeDtypeStruct((n1, 4, d // 8), jnp.int32),
            ],
            compiler_params=pltpu.CompilerParams(
                dimension_semantics=("parallel",),
            ),
            cost_estimate=pl.CostEstimate(
                flops=2 * n2 * d * d + n2 * d,
                transcendentals=0,
                bytes_accessed=(2 * n2 * d + n2 * d + d * d) * 4,
            ),
        )(embedding_2, fc1_w, bias_2d, embedding_1)
    else:
        e2_pack = pl.pallas_call(
            _fc1_pack_kernel,
            grid=(pl.cdiv(n2, tn),),
            in_specs=[
                pl.BlockSpec((tn, d), lambda i: (i, 0)),
                pl.BlockSpec((d, d), lambda i: (0, 0)),
                pl.BlockSpec((1, d), lambda i: (0, 0)),
            ],
            out_specs=pl.BlockSpec((tn, 4, d // 8), lambda i: (i, 0, 0)),
            out_shape=jax.ShapeDtypeStruct((n2, 4, d // 8), jnp.int32),
            compiler_params=pltpu.CompilerParams(
                dimension_semantics=("parallel",),
            ),
        )(embedding_2, fc1_w, bias_2d)
        e1_pack = pl.pallas_call(
            _pack_kernel,
            grid=(pl.cdiv(n1, tn),),
            in_specs=[pl.BlockSpec((tn, d), lambda i: (i, 0))],
            out_specs=pl.BlockSpec((tn, 4, d // 8), lambda i: (i, 0, 0)),
            out_shape=jax.ShapeDtypeStruct((n1, 4, d // 8), jnp.int32),
            compiler_params=pltpu.CompilerParams(
                dimension_semantics=("parallel",),
            ),
        )(embedding_1)

    ones_mat = jnp.ones((d // 8, 128), jnp.bfloat16)

    n_cores = 2
    per_core_p = p // n_cores
    tile_p = per_core_p if per_core_p <= 8192 else 8192
    unroll = 256 if tile_p % 256 == 0 else 1
    ii = observed_anchors_p[:, 0]
    jj = observed_anchors_p[:, 1]
    num_tiles = per_core_p // tile_p
    ii3 = ii.reshape(n_cores * num_tiles, 1, tile_p)
    jj3 = jj.reshape(n_cores * num_tiles, 1, tile_p)

    dk = functools.partial(_dist_kernel, tile_p=tile_p, unroll=unroll)
    partials = pl.pallas_call(
        dk,
        grid=(n_cores, num_tiles),
        in_specs=[
            pl.BlockSpec((1, 1, tile_p),
                         lambda c, t, _nt=num_tiles: (c * _nt + t, 0, 0),
                         memory_space=pltpu.MemorySpace.SMEM),
            pl.BlockSpec((1, 1, tile_p),
                         lambda c, t, _nt=num_tiles: (c * _nt + t, 0, 0),
                         memory_space=pltpu.MemorySpace.SMEM),
            pl.BlockSpec((n1, 4, d // 8), lambda c, t: (0, 0, 0)),
            pl.BlockSpec((n2, 4, d // 8), lambda c, t: (0, 0, 0)),
            pl.BlockSpec((d // 8, 128), lambda c, t: (0, 0)),
        ],
        out_specs=pl.BlockSpec((1, 1, 1), lambda c, t: (c, 0, 0),
                               memory_space=pltpu.MemorySpace.SMEM),
        out_shape=jax.ShapeDtypeStruct((n_cores, 1, 1), jnp.float32),
        scratch_shapes=[
            pltpu.VMEM((tile_p // 8, 8, d // 8), jnp.float32),
            pltpu.VMEM((8, 128), jnp.float32),
        ],
        compiler_params=pltpu.CompilerParams(
            dimension_semantics=("parallel", "arbitrary"),
        ),
        cost_estimate=pl.CostEstimate(
            flops=4 * p * d,
            transcendentals=p,
            bytes_accessed=(n1 * d // 2 + n2 * d // 2 + 2 * p) * 4,
        ),
    )(ii3, jj3, e1_pack, e2_pack, ones_mat)

    return jnp.sum(partials) * jnp.float32(1.0 / p)


# speedup vs baseline: 1.7513x; 1.0114x over previous
import functools

import jax
import jax.numpy as jnp
from jax.experimental import pallas as pl
from jax.experimental.pallas import tpu as pltpu


def _fc1_pack_kernel(x_ref, w_ref, b_ref, o_ref):
    tn = x_ref.shape[0]
    x = x_ref[...].astype(jnp.bfloat16)
    w = w_ref[...].astype(jnp.bfloat16)
    acc = jax.lax.dot_general(
        x, w, (((1,), (1,)), ((), ())),
        preferred_element_type=jnp.float32)
    bf = (acc + b_ref[...]).astype(jnp.bfloat16)
    o_ref[...] = pltpu.bitcast(bf.reshape(tn, 8, bf.shape[1] // 8),
                               jnp.int32)


def _pack_kernel(x_ref, o_ref):
    tn = x_ref.shape[0]
    bf = x_ref[...].astype(jnp.bfloat16)
    o_ref[...] = pltpu.bitcast(bf.reshape(tn, 8, bf.shape[1] // 8),
                               jnp.int32)


def _prep_kernel(x2_ref, w_ref, b_ref, x1_ref, o2_ref, o1_ref):
    _fc1_pack_kernel(x2_ref, w_ref, b_ref, o2_ref)
    _pack_kernel(x1_ref, o1_ref)


def _dist_kernel(ii_ref, jj_ref, e1_ref, e2_ref, ones_ref, out_ref,
                 sq_ref, acc_ref, *, tile_p, unroll):
    t = pl.program_id(1)

    @pl.when(t == 0)
    def _init():
        acc_ref[...] = jnp.zeros_like(acc_ref)

    def chunk(cstep, carry):
        base = cstep * unroll
        loads = []
        for u in range(unroll):
            i = ii_ref[0, 0, base + u]
            j = jj_ref[0, 0, base + u]
            loads.append((e1_ref[i], e2_ref[j]))
        for u, (ai, bj) in enumerate(loads):
            a = pltpu.bitcast(ai, jnp.bfloat16)
            b = pltpu.bitcast(bj, jnp.bfloat16)
            d = (a - b).astype(jnp.float32)
            row = cstep * (unroll // 8) + u // 8
            sq_ref[row, u % 8, :] = jnp.sum(d * d, axis=0)
        return carry

    jax.lax.fori_loop(0, tile_p // unroll, chunk, jnp.int32(0))

    y = sq_ref[...].reshape(tile_p, sq_ref.shape[2]).astype(jnp.bfloat16)
    s = jax.lax.dot_general(
        y, ones_ref[...], (((1,), (0,)), ((), ())),
        preferred_element_type=jnp.float32)
    dist = jnp.sqrt(s).reshape(tile_p // 8, 8, 128)
    acc_ref[...] += jnp.sum(dist, axis=0)

    @pl.when(t == pl.num_programs(1) - 1)
    def _finalize():
        out_ref[0, 0, 0] = jnp.sum(acc_ref[...]) * jnp.float32(1.0 / 128.0)


def kernel(embedding_1, embedding_2, observed_anchors_p, fc1_w, fc1_b):
    n1, d = embedding_1.shape
    n2, _ = embedding_2.shape
    p = observed_anchors_p.shape[0]

    tile_n = 512
    tn = min(n1, n2, tile_n)
    bias_2d = fc1_b.reshape(1, d)
    if n1 == n2:
        e2_pack, e1_pack = pl.pallas_call(
            _prep_kernel,
            grid=(pl.cdiv(n2, tn),),
            in_specs=[
                pl.BlockSpec((tn, d), lambda i: (i, 0)),
                pl.BlockSpec((d, d), lambda i: (0, 0)),
                pl.BlockSpec((1, d), lambda i: (0, 0)),
                pl.BlockSpec((tn, d), lambda i: (i, 0)),
            ],
            out_specs=[
                pl.BlockSpec((tn, 4, d // 8), lambda i: (i, 0, 0)),
                pl.BlockSpec((tn, 4, d // 8), lambda i: (i, 0, 0)),
            ],
            out_shape=[
                jax.ShapeDtypeStruct((n2, 4, d // 8), jnp.int32),
                jax.ShapeDtypeStruct((n1, 4, d // 8), jnp.int32),
            ],
            compiler_params=pltpu.CompilerParams(
                dimension_semantics=("parallel",),
            ),
            cost_estimate=pl.CostEstimate(
                flops=2 * n2 * d * d + n2 * d,
                transcendentals=0,
                bytes_accessed=(2 * n2 * d + n2 * d + d * d) * 4,
            ),
        )(embedding_2, fc1_w, bias_2d, embedding_1)
    else:
        e2_pack = pl.pallas_call(
            _fc1_pack_kernel,
            grid=(pl.cdiv(n2, tn),),
            in_specs=[
                pl.BlockSpec((tn, d), lambda i: (i, 0)),
                pl.BlockSpec((d, d), lambda i: (0, 0)),
                pl.BlockSpec((1, d), lambda i: (0, 0)),
            ],
            out_specs=pl.BlockSpec((tn, 4, d // 8), lambda i: (i, 0, 0)),
            out_shape=jax.ShapeDtypeStruct((n2, 4, d // 8), jnp.int32),
            compiler_params=pltpu.CompilerParams(
                dimension_semantics=("parallel",),
            ),
        )(embedding_2, fc1_w, bias_2d)
        e1_pack = pl.pallas_call(
            _pack_kernel,
            grid=(pl.cdiv(n1, tn),),
            in_specs=[pl.BlockSpec((tn, d), lambda i: (i, 0))],
            out_specs=pl.BlockSpec((tn, 4, d // 8), lambda i: (i, 0, 0)),
            out_shape=jax.ShapeDtypeStruct((n1, 4, d // 8), jnp.int32),
            compiler_params=pltpu.CompilerParams(
                dimension_semantics=("parallel",),
            ),
        )(embedding_1)

    ones_mat = jnp.ones((d // 8, 128), jnp.bfloat16)

    n_cores = 2
    per_core_p = p // n_cores
    tile_p = per_core_p if per_core_p <= 8192 else 8192
    unroll = 512 if tile_p % 512 == 0 else 1
    ii = observed_anchors_p[:, 0]
    jj = observed_anchors_p[:, 1]
    num_tiles = per_core_p // tile_p
    ii3 = ii.reshape(n_cores * num_tiles, 1, tile_p)
    jj3 = jj.reshape(n_cores * num_tiles, 1, tile_p)

    dk = functools.partial(_dist_kernel, tile_p=tile_p, unroll=unroll)
    partials = pl.pallas_call(
        dk,
        grid=(n_cores, num_tiles),
        in_specs=[
            pl.BlockSpec((1, 1, tile_p),
                         lambda c, t, _nt=num_tiles: (c * _nt + t, 0, 0),
                         memory_space=pltpu.MemorySpace.SMEM),
            pl.BlockSpec((1, 1, tile_p),
                         lambda c, t, _nt=num_tiles: (c * _nt + t, 0, 0),
                         memory_space=pltpu.MemorySpace.SMEM),
            pl.BlockSpec((n1, 4, d // 8), lambda c, t: (0, 0, 0)),
            pl.BlockSpec((n2, 4, d // 8), lambda c, t: (0, 0, 0)),
            pl.BlockSpec((d // 8, 128), lambda c, t: (0, 0)),
        ],
        out_specs=pl.BlockSpec((1, 1, 1), lambda c, t: (c, 0, 0),
                               memory_space=pltpu.MemorySpace.SMEM),
        out_shape=jax.ShapeDtypeStruct((n_cores, 1, 1), jnp.float32),
        scratch_shapes=[
            pltpu.VMEM((tile_p // 8, 8, d // 8), jnp.float32),
            pltpu.VMEM((8, 128), jnp.float32),
        ],
        compiler_params=pltpu.CompilerParams(
            dimension_semantics=("parallel", "arbitrary"),
        ),
        cost_estimate=pl.CostEstimate(
            flops=4 * p * d,
            transcendentals=p,
            bytes_accessed=(n1 * d // 2 + n2 * d // 2 + 2 * p) * 4,
        ),
    )(ii3, jj3, e1_pack, e2_pack, ones_mat)

    return jnp.sum(partials) * jnp.float32(1.0 / p)


# speedup vs baseline: 1.7603x; 1.0051x over previous
import functools

import jax
import jax.numpy as jnp
from jax.experimental import pallas as pl
from jax.experimental.pallas import tpu as pltpu


def _fc1_pack_kernel(x_ref, w_ref, b_ref, o_ref):
    tn = x_ref.shape[0]
    x = x_ref[...].astype(jnp.bfloat16)
    w = w_ref[...].astype(jnp.bfloat16)
    acc = jax.lax.dot_general(
        x, w, (((1,), (1,)), ((), ())),
        preferred_element_type=jnp.float32)
    bf = (acc + b_ref[...]).astype(jnp.bfloat16)
    o_ref[...] = pltpu.bitcast(bf.reshape(tn, 8, bf.shape[1] // 8),
                               jnp.int32)


def _pack_kernel(x_ref, o_ref):
    tn = x_ref.shape[0]
    bf = x_ref[...].astype(jnp.bfloat16)
    o_ref[...] = pltpu.bitcast(bf.reshape(tn, 8, bf.shape[1] // 8),
                               jnp.int32)


def _prep_kernel(x2_ref, w_ref, b_ref, x1_ref, o2_ref, o1_ref):
    _fc1_pack_kernel(x2_ref, w_ref, b_ref, o2_ref)
    _pack_kernel(x1_ref, o1_ref)


def _dist_kernel(ii_ref, jj_ref, e1_ref, e2_ref, ones_ref, out_ref,
                 sq_ref, acc_ref, *, tile_p, unroll):
    t = pl.program_id(1)

    @pl.when(t == 0)
    def _init():
        acc_ref[...] = jnp.zeros_like(acc_ref)

    def chunk(cstep, carry):
        base = cstep * unroll
        loads = []
        for u in range(unroll):
            i = ii_ref[0, 0, base + u]
            j = jj_ref[0, 0, base + u]
            loads.append((e1_ref[i], e2_ref[j]))
        for u, (ai, bj) in enumerate(loads):
            a = pltpu.bitcast(ai, jnp.bfloat16)
            b = pltpu.bitcast(bj, jnp.bfloat16)
            d = (a - b).astype(jnp.float32)
            row = cstep * (unroll // 8) + u // 8
            sq_ref[row, u % 8, :] = jnp.sum(d * d, axis=0)
        return carry

    jax.lax.fori_loop(0, tile_p // unroll, chunk, jnp.int32(0))

    y = sq_ref[...].reshape(tile_p, sq_ref.shape[2]).astype(jnp.bfloat16)
    s = jax.lax.dot_general(
        y, ones_ref[...], (((1,), (0,)), ((), ())),
        preferred_element_type=jnp.float32)
    dist = jnp.sqrt(s).reshape(tile_p // 8, 8, 128)
    acc_ref[...] += jnp.sum(dist, axis=0)

    @pl.when(t == pl.num_programs(1) - 1)
    def _finalize():
        out_ref[0, 0, 0] = jnp.sum(acc_ref[...]) * jnp.float32(1.0 / 128.0)


def kernel(embedding_1, embedding_2, observed_anchors_p, fc1_w, fc1_b):
    n1, d = embedding_1.shape
    n2, _ = embedding_2.shape
    p = observed_anchors_p.shape[0]

    tile_n = 512
    tn = min(n1, n2, tile_n)
    bias_2d = fc1_b.reshape(1, d)
    if n1 == n2:
        e2_pack, e1_pack = pl.pallas_call(
            _prep_kernel,
            grid=(pl.cdiv(n2, tn),),
            in_specs=[
                pl.BlockSpec((tn, d), lambda i: (i, 0)),
                pl.BlockSpec((d, d), lambda i: (0, 0)),
                pl.BlockSpec((1, d), lambda i: (0, 0)),
                pl.BlockSpec((tn, d), lambda i: (i, 0)),
            ],
            out_specs=[
                pl.BlockSpec((tn, 4, d // 8), lambda i: (i, 0, 0)),
                pl.BlockSpec((tn, 4, d // 8), lambda i: (i, 0, 0)),
            ],
            out_shape=[
                jax.ShapeDtypeStruct((n2, 4, d // 8), jnp.int32),
                jax.ShapeDtypeStruct((n1, 4, d // 8), jnp.int32),
            ],
            compiler_params=pltpu.CompilerParams(
                dimension_semantics=("parallel",),
            ),
            cost_estimate=pl.CostEstimate(
                flops=2 * n2 * d * d + n2 * d,
                transcendentals=0,
                bytes_accessed=(2 * n2 * d + n2 * d + d * d) * 4,
            ),
        )(embedding_2, fc1_w, bias_2d, embedding_1)
    else:
        e2_pack = pl.pallas_call(
            _fc1_pack_kernel,
            grid=(pl.cdiv(n2, tn),),
            in_specs=[
                pl.BlockSpec((tn, d), lambda i: (i, 0)),
                pl.BlockSpec((d, d), lambda i: (0, 0)),
                pl.BlockSpec((1, d), lambda i: (0, 0)),
            ],
            out_specs=pl.BlockSpec((tn, 4, d // 8), lambda i: (i, 0, 0)),
            out_shape=jax.ShapeDtypeStruct((n2, 4, d // 8), jnp.int32),
            compiler_params=pltpu.CompilerParams(
                dimension_semantics=("parallel",),
            ),
        )(embedding_2, fc1_w, bias_2d)
        e1_pack = pl.pallas_call(
            _pack_kernel,
            grid=(pl.cdiv(n1, tn),),
            in_specs=[pl.BlockSpec((tn, d), lambda i: (i, 0))],
            out_specs=pl.BlockSpec((tn, 4, d // 8), lambda i: (i, 0, 0)),
            out_shape=jax.ShapeDtypeStruct((n1, 4, d // 8), jnp.int32),
            compiler_params=pltpu.CompilerParams(
                dimension_semantics=("parallel",),
            ),
        )(embedding_1)

    ones_mat = jnp.ones((d // 8, 128), jnp.bfloat16)

    n_cores = 2
    per_core_p = p // n_cores
    tile_p = per_core_p if per_core_p <= 8192 else 8192
    unroll = 1024 if tile_p % 1024 == 0 else 1
    ii = observed_anchors_p[:, 0]
    jj = observed_anchors_p[:, 1]
    num_tiles = per_core_p // tile_p
    ii3 = ii.reshape(n_cores * num_tiles, 1, tile_p)
    jj3 = jj.reshape(n_cores * num_tiles, 1, tile_p)

    dk = functools.partial(_dist_kernel, tile_p=tile_p, unroll=unroll)
    partials = pl.pallas_call(
        dk,
        grid=(n_cores, num_tiles),
        in_specs=[
            pl.BlockSpec((1, 1, tile_p),
                         lambda c, t, _nt=num_tiles: (c * _nt + t, 0, 0),
                         memory_space=pltpu.MemorySpace.SMEM),
            pl.BlockSpec((1, 1, tile_p),
                         lambda c, t, _nt=num_tiles: (c * _nt + t, 0, 0),
                         memory_space=pltpu.MemorySpace.SMEM),
            pl.BlockSpec((n1, 4, d // 8), lambda c, t: (0, 0, 0)),
            pl.BlockSpec((n2, 4, d // 8), lambda c, t: (0, 0, 0)),
            pl.BlockSpec((d // 8, 128), lambda c, t: (0, 0)),
        ],
        out_specs=pl.BlockSpec((1, 1, 1), lambda c, t: (c, 0, 0),
                               memory_space=pltpu.MemorySpace.SMEM),
        out_shape=jax.ShapeDtypeStruct((n_cores, 1, 1), jnp.float32),
        scratch_shapes=[
            pltpu.VMEM((tile_p // 8, 8, d // 8), jnp.float32),
            pltpu.VMEM((8, 128), jnp.float32),
        ],
        compiler_params=pltpu.CompilerParams(
            dimension_semantics=("parallel", "arbitrary"),
        ),
        cost_estimate=pl.CostEstimate(
            flops=4 * p * d,
            transcendentals=p,
            bytes_accessed=(n1 * d // 2 + n2 * d // 2 + 2 * p) * 4,
        ),
    )(ii3, jj3, e1_pack, e2_pack, ones_mat)

    return jnp.sum(partials) * jnp.float32(1.0 / p)
